# Initial kernel scaffold; baseline (speedup 1.0000x reference)
import jax
import jax.numpy as jnp
from jax.experimental import pallas as pl


def kernel(node_feats, edge_feats, edge_index, Wpn, bpn, Wpe1, bpe1, Wpe2, bpe2, Wet, bet, Wm1, bm1, Wm2, bm2, Wle, ble, Wpn2, bpn2, Wlm1, blm1, Wlm2, blm2, bn_g, bn_b):
    raise NotImplementedError("write your pallas kernel here")


# speedup vs baseline: 1.4739x; 1.4739x over previous
import functools

import jax
import jax.numpy as jnp
from jax.experimental import pallas as pl
from jax.experimental.pallas import tpu as pltpu


def _lrelu(x):
    return jnp.where(x > 0, x, 0.01 * x)


def _node_dense_body(x_ref, wpn_ref, bpn_ref, wa_ref, wlv_ref, hv_ref, p_ref, lv_ref):
    x = x_ref[...]
    hv = _lrelu(jnp.dot(x, wpn_ref[...], preferred_element_type=jnp.float32)
                + bpn_ref[...])
    hv_ref[...] = hv
    p_ref[...] = jnp.dot(x, wa_ref[...], preferred_element_type=jnp.float32)
    lv_ref[...] = jnp.dot(hv, wlv_ref[...], preferred_element_type=jnp.float32)


def _node_dense(node_feats, Wpn, bpn, Wpe1_hi, wpe2_hi):
    N, Dn = node_feats.shape
    G = Wpn.shape[1]
    BN = 1000
    grid = (N // BN,)
    return pl.pallas_call(
        _node_dense_body,
        grid=grid,
        in_specs=[
            pl.BlockSpec((BN, Dn), lambda i: (i, 0)),
            pl.BlockSpec((Dn, G), lambda i: (0, 0)),
            pl.BlockSpec((G,), lambda i: (0,)),
            pl.BlockSpec((Dn, G), lambda i: (0, 0)),
            pl.BlockSpec((G, 1), lambda i: (0, 0)),
        ],
        out_specs=[
            pl.BlockSpec((BN, G), lambda i: (i, 0)),
            pl.BlockSpec((BN, G), lambda i: (i, 0)),
            pl.BlockSpec((BN, 1), lambda i: (i, 0)),
        ],
        out_shape=[
            jax.ShapeDtypeStruct((N, G), jnp.float32),
            jax.ShapeDtypeStruct((N, G), jnp.float32),
            jax.ShapeDtypeStruct((N, 1), jnp.float32),
        ],
    )(node_feats, Wpn, bpn, Wpe1_hi, wpe2_hi)


def kernel(node_feats, edge_feats, edge_index, Wpn, bpn, Wpe1, bpe1, Wpe2,
           bpe2, Wet, bet, Wm1, bm1, Wm2, bm2, Wle, ble, Wpn2, bpn2, Wlm1,
           blm1, Wlm2, blm2, bn_g, bn_b):
    return _impl(node_feats, edge_feats, edge_index, Wpn, bpn, Wpe1, bpe1,
                     Wpe2, bpe2, Wet, bet, Wm1, bm1, Wm2, bm2, Wle, ble, Wpn2,
                     bpn2, Wlm1, blm1, Wlm2, blm2, bn_g, bn_b)


def _impl(node_feats, edge_feats, edge_index, Wpn, bpn, Wpe1, bpe1, Wpe2,
          bpe2, Wet, bet, Wm1, bm1, Wm2, bm2, Wle, ble, Wpn2, bpn2, Wlm1,
          blm1, Wlm2, blm2, bn_g, bn_b):
    src = edge_index[0]
    dst = edge_index[1]
    N = node_feats.shape[0]
    Dn = node_feats.shape[1]
    G = Wpn.shape[1]
    lrelu = _lrelu

    hv_new, P, lv = _node_dense(node_feats, Wpn, bpn, Wpe1[:Dn], Wpe2[:G])
    lv = lv[:, 0]

    he1 = lrelu(P[src] + edge_feats @ Wpe1[Dn:] + bpe1)
    logits = lrelu(lv[dst] + he1 @ Wpe2[G:, 0] + bpe2[0])
    ex = jnp.exp(logits)
    S = jax.ops.segment_sum(ex, dst, num_segments=N)
    rS = 1.0 / (S + 1e-16)
    ews = (he1 @ Wet + bet) * ex[:, None]
    c = jax.ops.segment_sum(ews, dst, num_segments=N) * rS[:, None]
    ctx = jax.nn.elu(c)
    h = jax.nn.relu(ctx @ Wm1[:G] + hv_new @ Wm1[G:] + bm1)
    nf = jax.nn.relu(h @ Wm2 + bm2)

    u = nf @ Wle[:G, 0]
    v = nf @ Wle[G:, 0]
    logits2 = lrelu(u[dst] + v[src] + ble[0])
    ex2 = jnp.exp(logits2)
    S2 = jax.ops.segment_sum(ex2, dst, num_segments=N)
    hv2 = nf @ Wpn2 + bpn2
    t_raw = jax.ops.segment_sum(hv2[src] * ex2[:, None], dst, num_segments=N)
    c2 = t_raw / (S2[:, None] + 1e-16)
    ctx2 = jax.nn.elu(c2)
    h2 = jax.nn.relu(ctx2 @ Wlm1[:G] + nf @ Wlm1[G:] + blm1)
    h2 = jax.nn.relu(h2 @ Wlm2 + blm2)
    mu = jnp.mean(h2, axis=0)
    var = jnp.var(h2, axis=0)
    h2 = (h2 - mu) / jnp.sqrt(var + 1e-5) * bn_g + bn_b
    return nf + h2


# speedup vs baseline: 4.8692x; 3.3036x over previous
import functools

import jax
import jax.numpy as jnp
from jax import lax
from jax.experimental import pallas as pl
from jax.experimental.pallas import tpu as pltpu
from jax.experimental.pallas import tpu_sc as plsc

N = 10000
E = 320000
DN = 128
DE = 16
G = 256
H = G // 2
CH = 128
NCHUNK = E // CH
NW = 32
ROWS_PER_TILE = N // 16


def _lrelu(x):
    return jnp.where(x > 0, x, 0.01 * x)


def _elu(x):
    return jnp.where(x > 0, x, jnp.exp(x) - 1.0)


def _k1_body(x_ref, wpn_ref, bpn_ref, wa_ref, wlv_ref, hv_ref, p_ref, lv_ref):
    x = x_ref[...]
    hv = _lrelu(jnp.dot(x, wpn_ref[...], preferred_element_type=jnp.float32)
                + bpn_ref[...])
    hv_ref[...] = hv
    p_ref[...] = jnp.dot(x, wa_ref[...], preferred_element_type=jnp.float32)
    lv_ref[...] = jnp.dot(hv, wlv_ref[...], preferred_element_type=jnp.float32)


def _k1(node_feats, Wpn, bpn, Wpe1_hi, wpe2_hi):
    BN = 1000
    return pl.pallas_call(
        _k1_body,
        grid=(N // BN,),
        in_specs=[
            pl.BlockSpec((BN, DN), lambda i: (i, 0)),
            pl.BlockSpec((DN, G), lambda i: (0, 0)),
            pl.BlockSpec((G,), lambda i: (0,)),
            pl.BlockSpec((DN, G), lambda i: (0, 0)),
            pl.BlockSpec((G, 1), lambda i: (0, 0)),
        ],
        out_specs=[
            pl.BlockSpec((BN, G), lambda i: (i, 0)),
            pl.BlockSpec((BN, G), lambda i: (i, 0)),
            pl.BlockSpec((BN, 1), lambda i: (i, 0)),
        ],
        out_shape=[
            jax.ShapeDtypeStruct((N, G), jnp.float32),
            jax.ShapeDtypeStruct((N, G), jnp.float32),
            jax.ShapeDtypeStruct((N, 1), jnp.float32),
        ],
    )(node_feats, Wpn, bpn, Wpe1_hi, wpe2_hi)


def _sc_gather_body(table, t1, t2, ridx, i1, i2,
                    rows_out, s1_out, s2_out,
                    idxr_v, idx1_v, idx2_v, rows_v, s1_v, s2_v, sem):
    c = lax.axis_index("c")
    s = lax.axis_index("s")
    wid = s * 2 + c
    n_iter = 78 + jnp.where(wid < (NCHUNK - 78 * NW), 1, 0)

    def body(i, carry):
        ch = wid + i * NW
        pltpu.sync_copy(ridx.at[ch], idxr_v)
        pltpu.sync_copy(i1.at[ch], idx1_v)
        pltpu.sync_copy(i2.at[ch], idx2_v)
        pltpu.async_copy(table.at[idxr_v], rows_v, sem).wait()
        pltpu.sync_copy(rows_v, rows_out.at[pl.ds(ch * CH, CH)])
        pltpu.async_copy(t1.at[idx1_v], s1_v, sem).wait()
        pltpu.sync_copy(s1_v, s1_out.at[ch])
        pltpu.async_copy(t2.at[idx2_v], s2_v, sem).wait()
        pltpu.sync_copy(s2_v, s2_out.at[ch])
        return carry

    lax.fori_loop(0, n_iter, body, 0)


def _sc_gather(table, t1, t2, ridx2d, i12d, i22d):
    mesh = plsc.VectorSubcoreMesh(core_axis_name="c", subcore_axis_name="s")
    f = functools.partial(
        pl.kernel,
        mesh=mesh,
        out_type=[
            jax.ShapeDtypeStruct((E, G), jnp.float32),
            jax.ShapeDtypeStruct((NCHUNK, CH), jnp.float32),
            jax.ShapeDtypeStruct((NCHUNK, CH), jnp.float32),
        ],
        scratch_types=[
            pltpu.VMEM((CH,), jnp.int32),
            pltpu.VMEM((CH,), jnp.int32),
            pltpu.VMEM((CH,), jnp.int32),
            pltpu.VMEM((CH, G), jnp.float32),
            pltpu.VMEM((CH,), jnp.float32),
            pltpu.VMEM((CH,), jnp.float32),
            pltpu.SemaphoreType.DMA,
        ],
    )(_sc_gather_body)
    return f(table, t1, t2, ridx2d, i12d, i22d)


def _sc_scatter_body(rows_a, rows_b, exv, didx, z2, z1,
                     out_a, out_b, s_out,
                     idx_v, rows_v, ex_v, acc_sh, s_sh, sem):
    c = lax.axis_index("c")
    s = lax.axis_index("s")
    wid = s * 2 + c

    @pl.when(s == 0)
    def _zero():
        pltpu.sync_copy(z2, acc_sh)

    @pl.when(jnp.logical_and(s == 0, c == 0))
    def _zero_s():
        pltpu.sync_copy(z1, s_sh)

    plsc.subcore_barrier()

    n_iter = (NCHUNK // 16) + jnp.where(s < (NCHUNK - (NCHUNK // 16) * 16),
                                        1, 0)

    def body(i, carry):
        ch = s + i * 16
        pltpu.sync_copy(didx.at[ch], idx_v)

        @pl.when(c == 0)
        def _rows_a():
            pltpu.sync_copy(rows_a.at[pl.ds(ch * CH, CH)], rows_v)

        @pl.when(c == 1)
        def _rows_b():
            pltpu.sync_copy(rows_b.at[pl.ds(ch * CH, CH)], rows_v)

        pltpu.sync_copy(rows_v, acc_sh.at[idx_v], add=True)

        @pl.when(c == 0)
        def _s_add():
            pltpu.sync_copy(exv.at[ch], ex_v)
            pltpu.sync_copy(ex_v, s_sh.at[idx_v], add=True)

        return carry

    lax.fori_loop(0, n_iter, body, 0)
    plsc.subcore_barrier()

    @pl.when(jnp.logical_and(s == 0, c == 0))
    def _wb_a():
        pltpu.sync_copy(acc_sh, out_a)

    @pl.when(jnp.logical_and(s == 0, c == 1))
    def _wb_b():
        pltpu.sync_copy(acc_sh, out_b)

    @pl.when(jnp.logical_and(s == 0, c == 0))
    def _wb_s():
        pltpu.sync_copy(s_sh, s_out)


def _sc_scatter(rows_a, rows_b, ex2d, didx2d, z2, z1):
    mesh = plsc.VectorSubcoreMesh(core_axis_name="c", subcore_axis_name="s")
    f = functools.partial(
        pl.kernel,
        mesh=mesh,
        out_type=[
            jax.ShapeDtypeStruct((N, H), jnp.float32),
            jax.ShapeDtypeStruct((N, H), jnp.float32),
            jax.ShapeDtypeStruct((N,), jnp.float32),
        ],
        scratch_types=[
            pltpu.VMEM((CH,), jnp.int32),
            pltpu.VMEM((CH, H), jnp.float32),
            pltpu.VMEM((CH,), jnp.float32),
            pltpu.VMEM_SHARED((N, H), jnp.float32),
            pltpu.VMEM_SHARED((N,), jnp.float32),
            pltpu.SemaphoreType.DMA,
        ],
    )(_sc_scatter_body)
    return f(rows_a, rows_b, ex2d, didx2d, z2, z1)


def _k3_body(g_ref, ef_ref, ld_ref, wlo_ref, bpe1_ref, wlog_ref, bpe2_ref,
             wet_ref, bet_ref, ewa_ref, ewb_ref, ex_ref):
    he1 = _lrelu(g_ref[...]
                 + jnp.dot(ef_ref[...], wlo_ref[...],
                           preferred_element_type=jnp.float32)
                 + bpe1_ref[...])
    eb = jnp.dot(he1, wlog_ref[...], preferred_element_type=jnp.float32)
    ex = jnp.exp(_lrelu(ld_ref[...] + eb + bpe2_ref[...]))
    ews = (jnp.dot(he1, wet_ref[...], preferred_element_type=jnp.float32)
           + bet_ref[...]) * ex
    ewa_ref[...] = ews[:, :H]
    ewb_ref[...] = ews[:, H:]
    ex_ref[...] = ex


def _k3(g, ef, ld, Wpe1_lo, bpe1, wpe2_lo, bpe2, Wet, bet):
    BE = 2000
    return pl.pallas_call(
        _k3_body,
        grid=(E // BE,),
        in_specs=[
            pl.BlockSpec((BE, G), lambda i: (i, 0)),
            pl.BlockSpec((BE, DE), lambda i: (i, 0)),
            pl.BlockSpec((BE, 1), lambda i: (i, 0)),
            pl.BlockSpec((DE, G), lambda i: (0, 0)),
            pl.BlockSpec((G,), lambda i: (0,)),
            pl.BlockSpec((G, 1), lambda i: (0, 0)),
            pl.BlockSpec((1,), lambda i: (0,)),
            pl.BlockSpec((G, G), lambda i: (0, 0)),
            pl.BlockSpec((G,), lambda i: (0,)),
        ],
        out_specs=[
            pl.BlockSpec((BE, H), lambda i: (i, 0)),
            pl.BlockSpec((BE, H), lambda i: (i, 0)),
            pl.BlockSpec((BE, 1), lambda i: (i, 0)),
        ],
        out_shape=[
            jax.ShapeDtypeStruct((E, H), jnp.float32),
            jax.ShapeDtypeStruct((E, H), jnp.float32),
            jax.ShapeDtypeStruct((E, 1), jnp.float32),
        ],
    )(g, ef, ld, Wpe1_lo, bpe1, wpe2_lo, bpe2, Wet, bet)


def _k5_body(ca_ref, cb_ref, s_ref, hv_ref, wm1_ref, bm1_ref, wm2_ref,
             bm2_ref, wle_ref, wpn2_ref, bpn2_ref,
             nf_ref, u_ref, v_ref, hv2_ref):
    rs = 1.0 / (s_ref[...] + 1e-16)
    ctxa = _elu(ca_ref[...] * rs)
    ctxb = _elu(cb_ref[...] * rs)
    wm1 = wm1_ref[...]
    h = jax.nn.relu(
        jnp.dot(ctxa, wm1[:H], preferred_element_type=jnp.float32)
        + jnp.dot(ctxb, wm1[H:G], preferred_element_type=jnp.float32)
        + jnp.dot(hv_ref[...], wm1[G:], preferred_element_type=jnp.float32)
        + bm1_ref[...])
    nf = jax.nn.relu(jnp.dot(h, wm2_ref[...],
                             preferred_element_type=jnp.float32) + bm2_ref[...])
    nf_ref[...] = nf
    wle = wle_ref[...]
    u_ref[...] = jnp.dot(nf, wle[:G], preferred_element_type=jnp.float32)
    v_ref[...] = jnp.dot(nf, wle[G:], preferred_element_type=jnp.float32)
    hv2_ref[...] = jnp.dot(nf, wpn2_ref[...],
                           preferred_element_type=jnp.float32) + bpn2_ref[...]


def _k5(craw_a, craw_b, S, hv_new, Wm1, bm1, Wm2, bm2, Wle, Wpn2, bpn2):
    BN = 1000
    return pl.pallas_call(
        _k5_body,
        grid=(N // BN,),
        in_specs=[
            pl.BlockSpec((BN, H), lambda i: (i, 0)),
            pl.BlockSpec((BN, H), lambda i: (i, 0)),
            pl.BlockSpec((BN, 1), lambda i: (i, 0)),
            pl.BlockSpec((BN, G), lambda i: (i, 0)),
            pl.BlockSpec((2 * G, G), lambda i: (0, 0)),
            pl.BlockSpec((G,), lambda i: (0,)),
            pl.BlockSpec((G, G), lambda i: (0, 0)),
            pl.BlockSpec((G,), lambda i: (0,)),
            pl.BlockSpec((2 * G, 1), lambda i: (0, 0)),
            pl.BlockSpec((G, G), lambda i: (0, 0)),
            pl.BlockSpec((G,), lambda i: (0,)),
        ],
        out_specs=[
            pl.BlockSpec((BN, G), lambda i: (i, 0)),
            pl.BlockSpec((BN, 1), lambda i: (i, 0)),
            pl.BlockSpec((BN, 1), lambda i: (i, 0)),
            pl.BlockSpec((BN, G), lambda i: (i, 0)),
        ],
        out_shape=[
            jax.ShapeDtypeStruct((N, G), jnp.float32),
            jax.ShapeDtypeStruct((N, 1), jnp.float32),
            jax.ShapeDtypeStruct((N, 1), jnp.float32),
            jax.ShapeDtypeStruct((N, G), jnp.float32),
        ],
    )(craw_a, craw_b, S, hv_new, Wm1, bm1, Wm2, bm2, Wle, Wpn2, bpn2)


def _k6b_body(g2_ref, ud_ref, vs_ref, ble_ref, ga_ref, gb_ref, ex2_ref):
    ex2 = jnp.exp(_lrelu(ud_ref[...] + vs_ref[...] + ble_ref[...]))
    g2s = g2_ref[...] * ex2
    ga_ref[...] = g2s[:, :H]
    gb_ref[...] = g2s[:, H:]
    ex2_ref[...] = ex2


def _k6b(g2, ud, vs, ble):
    BE = 4000
    return pl.pallas_call(
        _k6b_body,
        grid=(E // BE,),
        in_specs=[
            pl.BlockSpec((BE, G), lambda i: (i, 0)),
            pl.BlockSpec((BE, 1), lambda i: (i, 0)),
            pl.BlockSpec((BE, 1), lambda i: (i, 0)),
            pl.BlockSpec((1,), lambda i: (0,)),
        ],
        out_specs=[
            pl.BlockSpec((BE, H), lambda i: (i, 0)),
            pl.BlockSpec((BE, H), lambda i: (i, 0)),
            pl.BlockSpec((BE, 1), lambda i: (i, 0)),
        ],
        out_shape=[
            jax.ShapeDtypeStruct((E, H), jnp.float32),
            jax.ShapeDtypeStruct((E, H), jnp.float32),
            jax.ShapeDtypeStruct((E, 1), jnp.float32),
        ],
    )(g2, ud, vs, ble)


def _k7_body(ta_ref, tb_ref, s2_ref, nf_ref, wlm1_ref, blm1_ref, wlm2_ref,
             blm2_ref, h2_ref, sum_ref, sq_ref):
    i = pl.program_id(0)
    rs2 = 1.0 / (s2_ref[...] + 1e-16)
    ctxa = _elu(ta_ref[...] * rs2)
    ctxb = _elu(tb_ref[...] * rs2)
    wlm1 = wlm1_ref[...]
    h2 = jax.nn.relu(
        jnp.dot(ctxa, wlm1[:H], preferred_element_type=jnp.float32)
        + jnp.dot(ctxb, wlm1[H:G], preferred_element_type=jnp.float32)
        + jnp.dot(nf_ref[...], wlm1[G:], preferred_element_type=jnp.float32)
        + blm1_ref[...])
    h2 = jax.nn.relu(jnp.dot(h2, wlm2_ref[...],
                             preferred_element_type=jnp.float32) + blm2_ref[...])
    h2_ref[...] = h2

    @pl.when(i == 0)
    def _init():
        sum_ref[...] = jnp.zeros_like(sum_ref)
        sq_ref[...] = jnp.zeros_like(sq_ref)

    bs = jnp.sum(h2, axis=0, keepdims=True)
    bq = jnp.sum(h2 * h2, axis=0, keepdims=True)
    sum_ref[...] += jnp.broadcast_to(bs, sum_ref.shape)
    sq_ref[...] += jnp.broadcast_to(bq, sq_ref.shape)


def _k7(traw_a, traw_b, S2, nf2, Wlm1, blm1, Wlm2, blm2):
    BN = 1000
    return pl.pallas_call(
        _k7_body,
        grid=(N // BN,),
        in_specs=[
            pl.BlockSpec((BN, H), lambda i: (i, 0)),
            pl.BlockSpec((BN, H), lambda i: (i, 0)),
            pl.BlockSpec((BN, 1), lambda i: (i, 0)),
            pl.BlockSpec((BN, G), lambda i: (i, 0)),
            pl.BlockSpec((2 * G, G), lambda i: (0, 0)),
            pl.BlockSpec((G,), lambda i: (0,)),
            pl.BlockSpec((G, G), lambda i: (0, 0)),
            pl.BlockSpec((G,), lambda i: (0,)),
        ],
        out_specs=[
            pl.BlockSpec((BN, G), lambda i: (i, 0)),
            pl.BlockSpec((8, G), lambda i: (0, 0)),
            pl.BlockSpec((8, G), lambda i: (0, 0)),
        ],
        out_shape=[
            jax.ShapeDtypeStruct((N, G), jnp.float32),
            jax.ShapeDtypeStruct((8, G), jnp.float32),
            jax.ShapeDtypeStruct((8, G), jnp.float32),
        ],
    )(traw_a, traw_b, S2, nf2, Wlm1, blm1, Wlm2, blm2)


def _k8_body(h2_ref, sum_ref, sq_ref, nf_ref, g_ref, b_ref, out_ref):
    mu = sum_ref[0:1, :] * (1.0 / N)
    var = sq_ref[0:1, :] * (1.0 / N) - mu * mu
    inv = jax.lax.rsqrt(var + 1e-5)
    out_ref[...] = nf_ref[...] + (h2_ref[...] - mu) * inv * g_ref[...] + b_ref[...]


def _k8(h2, sums, sqs, nf2, bn_g, bn_b):
    BN = 1000
    gm = bn_g.reshape(1, G)
    bm = (bn_b - 0.0).reshape(1, G)
    return pl.pallas_call(
        _k8_body,
        grid=(N // BN,),
        in_specs=[
            pl.BlockSpec((BN, G), lambda i: (i, 0)),
            pl.BlockSpec((8, G), lambda i: (0, 0)),
            pl.BlockSpec((8, G), lambda i: (0, 0)),
            pl.BlockSpec((BN, G), lambda i: (i, 0)),
            pl.BlockSpec((1, G), lambda i: (0, 0)),
            pl.BlockSpec((1, G), lambda i: (0, 0)),
        ],
        out_specs=pl.BlockSpec((BN, G), lambda i: (i, 0)),
        out_shape=jax.ShapeDtypeStruct((N, G), jnp.float32),
    )(h2, sums, sqs, nf2, gm, bm)


_USE_SC_GATHER = True
_USE_SC_SCATTER = True


def kernel(node_feats, edge_feats, edge_index, Wpn, bpn, Wpe1, bpe1, Wpe2,
           bpe2, Wet, bet, Wm1, bm1, Wm2, bm2, Wle, ble, Wpn2, bpn2, Wlm1,
           blm1, Wlm2, blm2, bn_g, bn_b):
    src = edge_index[0]
    dst = edge_index[1]
    src2d = src.reshape(NCHUNK, CH)
    dst2d = dst.reshape(NCHUNK, CH)
    z2 = jnp.zeros((N, H), jnp.float32)
    z1 = jnp.zeros((N,), jnp.float32)

    hv_new, P, lv = _k1(node_feats, Wpn, bpn, Wpe1[:DN], Wpe2[:G])
    lv1 = lv[:, 0]

    if _USE_SC_GATHER:
        g1, ld2d, _ = _sc_gather(P, lv1, lv1, src2d, dst2d, dst2d)
        ld = ld2d.reshape(E, 1)
    else:
        g1 = P[src]
        ld = lv1[dst].reshape(E, 1)

    ewa, ewb, ex = _k3(g1, edge_feats, ld, Wpe1[DN:], bpe1, Wpe2[G:], bpe2,
                       Wet, bet)

    if _USE_SC_SCATTER:
        craw_a, craw_b, S = _sc_scatter(ewa, ewb, ex.reshape(NCHUNK, CH),
                                        dst2d, z2, z1)
    else:
        craw_a = jax.ops.segment_sum(ewa, dst, num_segments=N)
        craw_b = jax.ops.segment_sum(ewb, dst, num_segments=N)
        S = jax.ops.segment_sum(ex[:, 0], dst, num_segments=N)

    nf2, u, v, hv2 = _k5(craw_a, craw_b, S.reshape(N, 1), hv_new,
                         Wm1, bm1, Wm2, bm2, Wle, Wpn2, bpn2)

    if _USE_SC_GATHER:
        g2, ud2d, vs2d = _sc_gather(hv2, u[:, 0], v[:, 0], src2d, dst2d,
                                    src2d)
        ud, vs = ud2d.reshape(E, 1), vs2d.reshape(E, 1)
    else:
        g2 = hv2[src]
        ud, vs = u[dst], v[src]

    ga, gb, ex2 = _k6b(g2, ud, vs, ble)

    if _USE_SC_SCATTER:
        traw_a, traw_b, S2 = _sc_scatter(ga, gb, ex2.reshape(NCHUNK, CH),
                                         dst2d, z2, z1)
    else:
        traw_a = jax.ops.segment_sum(ga, dst, num_segments=N)
        traw_b = jax.ops.segment_sum(gb, dst, num_segments=N)
        S2 = jax.ops.segment_sum(ex2[:, 0], dst, num_segments=N)

    h2, sums, sqs = _k7(traw_a, traw_b, S2.reshape(N, 1), nf2,
                        Wlm1, blm1, Wlm2, blm2)
    return _k8(h2, sums, sqs, nf2, bn_g, bn_b)


# speedup vs baseline: 6.8721x; 1.4113x over previous
import functools

import jax
import jax.numpy as jnp
from jax import lax
from jax.experimental import pallas as pl
from jax.experimental.pallas import tpu as pltpu
from jax.experimental.pallas import tpu_sc as plsc

N = 10000
E = 320000
DN = 128
DE = 16
G = 256
H = G // 2
CH = 128
NCHUNK = E // CH
NW = 32
ROWS_PER_TILE = N // 16


def _lrelu(x):
    return jnp.where(x > 0, x, 0.01 * x)


def _elu(x):
    return jnp.where(x > 0, x, jnp.exp(x) - 1.0)


def _k1_body(x_ref, wpn_ref, bpn_ref, wa_ref, wlv_ref, hv_ref, p_ref, lv_ref):
    x = x_ref[...]
    hv = _lrelu(jnp.dot(x, wpn_ref[...], preferred_element_type=jnp.float32)
                + bpn_ref[...])
    hv_ref[...] = hv
    p_ref[...] = jnp.dot(x, wa_ref[...], preferred_element_type=jnp.float32)
    lv_ref[...] = jnp.dot(hv, wlv_ref[...], preferred_element_type=jnp.float32)


def _k1(node_feats, Wpn, bpn, Wpe1_hi, wpe2_hi):
    BN = 1000
    return pl.pallas_call(
        _k1_body,
        grid=(N // BN,),
        in_specs=[
            pl.BlockSpec((BN, DN), lambda i: (i, 0)),
            pl.BlockSpec((DN, G), lambda i: (0, 0)),
            pl.BlockSpec((G,), lambda i: (0,)),
            pl.BlockSpec((DN, G), lambda i: (0, 0)),
            pl.BlockSpec((G, 1), lambda i: (0, 0)),
        ],
        out_specs=[
            pl.BlockSpec((BN, G), lambda i: (i, 0)),
            pl.BlockSpec((BN, G), lambda i: (i, 0)),
            pl.BlockSpec((BN, 1), lambda i: (i, 0)),
        ],
        out_shape=[
            jax.ShapeDtypeStruct((N, G), jnp.float32),
            jax.ShapeDtypeStruct((N, G), jnp.float32),
            jax.ShapeDtypeStruct((N, 1), jnp.float32),
        ],
    )(node_feats, Wpn, bpn, Wpe1_hi, wpe2_hi)


def _sc_gather_body(table, t1, t2, ridx, i1, i2,
                    rows_out, s1_out, s2_out,
                    idxr_v, idx1_v, idx2_v, rows_v, s1_v, s2_v,
                    sem_i, sem_g, sem_w):
    c = lax.axis_index("c")
    s = lax.axis_index("s")
    wid = s * 2 + c
    base = (NCHUNK // NW) & ~1
    rem = NCHUNK - base * NW
    n_pairs = (base + jnp.where(wid < rem // 2, 2, 0)) // 2

    def chunk_of(i):
        return jnp.where(i < base, wid + i * NW,
                         base * NW + (i - base) * (rem // 2) + wid)

    def issue_idx(ch, slot):
        h0 = pltpu.async_copy(ridx.at[ch], idxr_v.at[slot], sem_i)
        h1 = pltpu.async_copy(i1.at[ch], idx1_v.at[slot], sem_i)
        h2 = pltpu.async_copy(i2.at[ch], idx2_v.at[slot], sem_i)
        return (h0, h1, h2)

    def issue_gather(slot):
        h0 = pltpu.async_copy(table.at[idxr_v.at[slot]], rows_v.at[slot],
                              sem_g)
        h1 = pltpu.async_copy(t1.at[idx1_v.at[slot]], s1_v.at[slot], sem_g)
        h2 = pltpu.async_copy(t2.at[idx2_v.at[slot]], s2_v.at[slot], sem_g)
        return (h0, h1, h2)

    def issue_write(ch, slot):
        h0 = pltpu.async_copy(rows_v.at[slot],
                              rows_out.at[pl.ds(ch * CH, CH)], sem_w)
        h1 = pltpu.async_copy(s1_v.at[slot], s1_out.at[ch], sem_w)
        h2 = pltpu.async_copy(s2_v.at[slot], s2_out.at[ch], sem_w)
        return (h0, h1, h2)

    def wait(hs):
        for h in hs:
            h.wait()

    def body(j, carry):
        ch0 = chunk_of(2 * j)
        ch1 = chunk_of(2 * j + 1)
        ia = issue_idx(ch0, 0)
        ib = issue_idx(ch1, 1)
        wait(ia)
        ga = issue_gather(0)
        wait(ib)
        gb = issue_gather(1)
        wait(ga)
        wa = issue_write(ch0, 0)
        wait(gb)
        wb = issue_write(ch1, 1)
        wait(wa)
        wait(wb)
        return carry

    lax.fori_loop(0, n_pairs, body, 0)


def _sc_gather(table, t1, t2, ridx2d, i12d, i22d):
    mesh = plsc.VectorSubcoreMesh(core_axis_name="c", subcore_axis_name="s")
    f = functools.partial(
        pl.kernel,
        mesh=mesh,
        out_type=[
            jax.ShapeDtypeStruct((E, G), jnp.float32),
            jax.ShapeDtypeStruct((NCHUNK, CH), jnp.float32),
            jax.ShapeDtypeStruct((NCHUNK, CH), jnp.float32),
        ],
        scratch_types=[
            pltpu.VMEM((2, CH), jnp.int32),
            pltpu.VMEM((2, CH), jnp.int32),
            pltpu.VMEM((2, CH), jnp.int32),
            pltpu.VMEM((2, CH, G), jnp.float32),
            pltpu.VMEM((2, CH), jnp.float32),
            pltpu.VMEM((2, CH), jnp.float32),
            pltpu.SemaphoreType.DMA,
            pltpu.SemaphoreType.DMA,
            pltpu.SemaphoreType.DMA,
        ],
    )(_sc_gather_body)
    return f(table, t1, t2, ridx2d, i12d, i22d)


def _sc_scatter_body(rows_a, rows_b, exv, didx, z2, z1,
                     out_a, out_b, s_out,
                     idx_v, rows_v, ex_v, acc_sh, s_sh, sem_l, sem_s):
    c = lax.axis_index("c")
    s = lax.axis_index("s")
    wid = s * 2 + c

    @pl.when(s == 0)
    def _zero():
        pltpu.sync_copy(z2, acc_sh)

    @pl.when(jnp.logical_and(s == 0, c == 0))
    def _zero_s():
        pltpu.sync_copy(z1, s_sh)

    plsc.subcore_barrier()

    base = (NCHUNK // 16) & ~1
    rem = NCHUNK - base * 16
    n_pairs = (base + jnp.where(s < rem // 2, 2, 0)) // 2

    def chunk_of(i):
        return jnp.where(i < base, s + i * 16,
                         base * 16 + (i - base) * (rem // 2) + s)

    def issue_load(ch, slot):
        h0 = pltpu.async_copy(didx.at[ch], idx_v.at[slot], sem_l)
        h2 = pltpu.async_copy(exv.at[ch], ex_v.at[slot], sem_l)
        hs = [h0, h2]
        @pl.when(c == 0)
        def _la():
            pltpu.async_copy(rows_a.at[pl.ds(ch * CH, CH)], rows_v.at[slot],
                             sem_l)
        @pl.when(c == 1)
        def _lb():
            pltpu.async_copy(rows_b.at[pl.ds(ch * CH, CH)], rows_v.at[slot],
                             sem_l)
        hr = pltpu.make_async_copy(rows_a.at[pl.ds(0, CH)], rows_v.at[slot],
                                   sem_l)
        return hs, hr

    def wait_load(hs_hr):
        hs, hr = hs_hr
        for h in hs:
            h.wait()
        hr.wait()

    def issue_scatter(slot):
        h0 = pltpu.async_copy(rows_v.at[slot], acc_sh.at[idx_v.at[slot]],
                              sem_s, add=True)
        hs = [h0]
        @pl.when(c == 0)
        def _sa():
            pltpu.async_copy(ex_v.at[slot], s_sh.at[idx_v.at[slot]], sem_s,
                             add=True)
        return hs

    def wait_scatter(hs):
        for h in hs:
            h.wait()
        @pl.when(c == 0)
        def _ws():
            pltpu.make_async_copy(exv.at[0], ex_v.at[0], sem_s).wait()

    def body(j, carry):
        ch0 = chunk_of(2 * j)
        ch1 = chunk_of(2 * j + 1)
        la = issue_load(ch0, 0)
        lb = issue_load(ch1, 1)
        wait_load(la)
        sa = issue_scatter(0)
        wait_load(lb)
        sb = issue_scatter(1)
        wait_scatter(sa)
        wait_scatter(sb)
        return carry

    lax.fori_loop(0, n_pairs, body, 0)
    plsc.subcore_barrier()

    @pl.when(jnp.logical_and(s == 0, c == 0))
    def _wb_a():
        pltpu.sync_copy(acc_sh, out_a)

    @pl.when(jnp.logical_and(s == 0, c == 1))
    def _wb_b():
        pltpu.sync_copy(acc_sh, out_b)

    @pl.when(jnp.logical_and(s == 0, c == 0))
    def _wb_s():
        pltpu.sync_copy(s_sh, s_out)


def _sc_scatter(rows_a, rows_b, ex2d, didx2d, z2, z1):
    mesh = plsc.VectorSubcoreMesh(core_axis_name="c", subcore_axis_name="s")
    f = functools.partial(
        pl.kernel,
        mesh=mesh,
        out_type=[
            jax.ShapeDtypeStruct((N, H), jnp.float32),
            jax.ShapeDtypeStruct((N, H), jnp.float32),
            jax.ShapeDtypeStruct((N,), jnp.float32),
        ],
        scratch_types=[
            pltpu.VMEM((2, CH), jnp.int32),
            pltpu.VMEM((2, CH, H), jnp.float32),
            pltpu.VMEM((2, CH), jnp.float32),
            pltpu.VMEM_SHARED((N, H), jnp.float32),
            pltpu.VMEM_SHARED((N,), jnp.float32),
            pltpu.SemaphoreType.DMA,
            pltpu.SemaphoreType.DMA,
        ],
    )(_sc_scatter_body)
    return f(rows_a, rows_b, ex2d, didx2d, z2, z1)


def _k3_body(g_ref, ef_ref, ld_ref, wlo_ref, bpe1_ref, wlog_ref, bpe2_ref,
             wet_ref, bet_ref, ewa_ref, ewb_ref, ex_ref):
    he1 = _lrelu(g_ref[...]
                 + jnp.dot(ef_ref[...], wlo_ref[...],
                           preferred_element_type=jnp.float32)
                 + bpe1_ref[...])
    eb = jnp.dot(he1, wlog_ref[...], preferred_element_type=jnp.float32)
    ex = jnp.exp(_lrelu(ld_ref[...] + eb + bpe2_ref[...]))
    ews = (jnp.dot(he1, wet_ref[...], preferred_element_type=jnp.float32)
           + bet_ref[...]) * ex
    ewa_ref[...] = ews[:, :H]
    ewb_ref[...] = ews[:, H:]
    ex_ref[...] = ex


def _k3(g, ef, ld, Wpe1_lo, bpe1, wpe2_lo, bpe2, Wet, bet):
    BE = 2000
    return pl.pallas_call(
        _k3_body,
        grid=(E // BE,),
        in_specs=[
            pl.BlockSpec((BE, G), lambda i: (i, 0)),
            pl.BlockSpec((BE, DE), lambda i: (i, 0)),
            pl.BlockSpec((BE, 1), lambda i: (i, 0)),
            pl.BlockSpec((DE, G), lambda i: (0, 0)),
            pl.BlockSpec((G,), lambda i: (0,)),
            pl.BlockSpec((G, 1), lambda i: (0, 0)),
            pl.BlockSpec((1,), lambda i: (0,)),
            pl.BlockSpec((G, G), lambda i: (0, 0)),
            pl.BlockSpec((G,), lambda i: (0,)),
        ],
        out_specs=[
            pl.BlockSpec((BE, H), lambda i: (i, 0)),
            pl.BlockSpec((BE, H), lambda i: (i, 0)),
            pl.BlockSpec((BE, 1), lambda i: (i, 0)),
        ],
        out_shape=[
            jax.ShapeDtypeStruct((E, H), jnp.float32),
            jax.ShapeDtypeStruct((E, H), jnp.float32),
            jax.ShapeDtypeStruct((E, 1), jnp.float32),
        ],
    )(g, ef, ld, Wpe1_lo, bpe1, wpe2_lo, bpe2, Wet, bet)


def _k5_body(ca_ref, cb_ref, s_ref, hv_ref, wm1_ref, bm1_ref, wm2_ref,
             bm2_ref, wle_ref, wpn2_ref, bpn2_ref,
             nf_ref, u_ref, v_ref, hv2_ref):
    rs = 1.0 / (s_ref[...] + 1e-16)
    ctxa = _elu(ca_ref[...] * rs)
    ctxb = _elu(cb_ref[...] * rs)
    wm1 = wm1_ref[...]
    h = jax.nn.relu(
        jnp.dot(ctxa, wm1[:H], preferred_element_type=jnp.float32)
        + jnp.dot(ctxb, wm1[H:G], preferred_element_type=jnp.float32)
        + jnp.dot(hv_ref[...], wm1[G:], preferred_element_type=jnp.float32)
        + bm1_ref[...])
    nf = jax.nn.relu(jnp.dot(h, wm2_ref[...],
                             preferred_element_type=jnp.float32) + bm2_ref[...])
    nf_ref[...] = nf
    wle = wle_ref[...]
    u_ref[...] = jnp.dot(nf, wle[:G], preferred_element_type=jnp.float32)
    v_ref[...] = jnp.dot(nf, wle[G:], preferred_element_type=jnp.float32)
    hv2_ref[...] = jnp.dot(nf, wpn2_ref[...],
                           preferred_element_type=jnp.float32) + bpn2_ref[...]


def _k5(craw_a, craw_b, S, hv_new, Wm1, bm1, Wm2, bm2, Wle, Wpn2, bpn2):
    BN = 1000
    return pl.pallas_call(
        _k5_body,
        grid=(N // BN,),
        in_specs=[
            pl.BlockSpec((BN, H), lambda i: (i, 0)),
            pl.BlockSpec((BN, H), lambda i: (i, 0)),
            pl.BlockSpec((BN, 1), lambda i: (i, 0)),
            pl.BlockSpec((BN, G), lambda i: (i, 0)),
            pl.BlockSpec((2 * G, G), lambda i: (0, 0)),
            pl.BlockSpec((G,), lambda i: (0,)),
            pl.BlockSpec((G, G), lambda i: (0, 0)),
            pl.BlockSpec((G,), lambda i: (0,)),
            pl.BlockSpec((2 * G, 1), lambda i: (0, 0)),
            pl.BlockSpec((G, G), lambda i: (0, 0)),
            pl.BlockSpec((G,), lambda i: (0,)),
        ],
        out_specs=[
            pl.BlockSpec((BN, G), lambda i: (i, 0)),
            pl.BlockSpec((BN, 1), lambda i: (i, 0)),
            pl.BlockSpec((BN, 1), lambda i: (i, 0)),
            pl.BlockSpec((BN, G), lambda i: (i, 0)),
        ],
        out_shape=[
            jax.ShapeDtypeStruct((N, G), jnp.float32),
            jax.ShapeDtypeStruct((N, 1), jnp.float32),
            jax.ShapeDtypeStruct((N, 1), jnp.float32),
            jax.ShapeDtypeStruct((N, G), jnp.float32),
        ],
    )(craw_a, craw_b, S, hv_new, Wm1, bm1, Wm2, bm2, Wle, Wpn2, bpn2)


def _k6b_body(g2_ref, ud_ref, vs_ref, ble_ref, ga_ref, gb_ref, ex2_ref):
    ex2 = jnp.exp(_lrelu(ud_ref[...] + vs_ref[...] + ble_ref[...]))
    g2s = g2_ref[...] * ex2
    ga_ref[...] = g2s[:, :H]
    gb_ref[...] = g2s[:, H:]
    ex2_ref[...] = ex2


def _k6b(g2, ud, vs, ble):
    BE = 4000
    return pl.pallas_call(
        _k6b_body,
        grid=(E // BE,),
        in_specs=[
            pl.BlockSpec((BE, G), lambda i: (i, 0)),
            pl.BlockSpec((BE, 1), lambda i: (i, 0)),
            pl.BlockSpec((BE, 1), lambda i: (i, 0)),
            pl.BlockSpec((1,), lambda i: (0,)),
        ],
        out_specs=[
            pl.BlockSpec((BE, H), lambda i: (i, 0)),
            pl.BlockSpec((BE, H), lambda i: (i, 0)),
            pl.BlockSpec((BE, 1), lambda i: (i, 0)),
        ],
        out_shape=[
            jax.ShapeDtypeStruct((E, H), jnp.float32),
            jax.ShapeDtypeStruct((E, H), jnp.float32),
            jax.ShapeDtypeStruct((E, 1), jnp.float32),
        ],
    )(g2, ud, vs, ble)


def _k7_body(ta_ref, tb_ref, s2_ref, nf_ref, wlm1_ref, blm1_ref, wlm2_ref,
             blm2_ref, h2_ref, sum_ref, sq_ref):
    i = pl.program_id(0)
    rs2 = 1.0 / (s2_ref[...] + 1e-16)
    ctxa = _elu(ta_ref[...] * rs2)
    ctxb = _elu(tb_ref[...] * rs2)
    wlm1 = wlm1_ref[...]
    h2 = jax.nn.relu(
        jnp.dot(ctxa, wlm1[:H], preferred_element_type=jnp.float32)
        + jnp.dot(ctxb, wlm1[H:G], preferred_element_type=jnp.float32)
        + jnp.dot(nf_ref[...], wlm1[G:], preferred_element_type=jnp.float32)
        + blm1_ref[...])
    h2 = jax.nn.relu(jnp.dot(h2, wlm2_ref[...],
                             preferred_element_type=jnp.float32) + blm2_ref[...])
    h2_ref[...] = h2

    @pl.when(i == 0)
    def _init():
        sum_ref[...] = jnp.zeros_like(sum_ref)
        sq_ref[...] = jnp.zeros_like(sq_ref)

    bs = jnp.sum(h2, axis=0, keepdims=True)
    bq = jnp.sum(h2 * h2, axis=0, keepdims=True)
    sum_ref[...] += jnp.broadcast_to(bs, sum_ref.shape)
    sq_ref[...] += jnp.broadcast_to(bq, sq_ref.shape)


def _k7(traw_a, traw_b, S2, nf2, Wlm1, blm1, Wlm2, blm2):
    BN = 1000
    return pl.pallas_call(
        _k7_body,
        grid=(N // BN,),
        in_specs=[
            pl.BlockSpec((BN, H), lambda i: (i, 0)),
            pl.BlockSpec((BN, H), lambda i: (i, 0)),
            pl.BlockSpec((BN, 1), lambda i: (i, 0)),
            pl.BlockSpec((BN, G), lambda i: (i, 0)),
            pl.BlockSpec((2 * G, G), lambda i: (0, 0)),
            pl.BlockSpec((G,), lambda i: (0,)),
            pl.BlockSpec((G, G), lambda i: (0, 0)),
            pl.BlockSpec((G,), lambda i: (0,)),
        ],
        out_specs=[
            pl.BlockSpec((BN, G), lambda i: (i, 0)),
            pl.BlockSpec((8, G), lambda i: (0, 0)),
            pl.BlockSpec((8, G), lambda i: (0, 0)),
        ],
        out_shape=[
            jax.ShapeDtypeStruct((N, G), jnp.float32),
            jax.ShapeDtypeStruct((8, G), jnp.float32),
            jax.ShapeDtypeStruct((8, G), jnp.float32),
        ],
    )(traw_a, traw_b, S2, nf2, Wlm1, blm1, Wlm2, blm2)


def _k8_body(h2_ref, sum_ref, sq_ref, nf_ref, g_ref, b_ref, out_ref):
    mu = sum_ref[0:1, :] * (1.0 / N)
    var = sq_ref[0:1, :] * (1.0 / N) - mu * mu
    inv = jax.lax.rsqrt(var + 1e-5)
    out_ref[...] = nf_ref[...] + (h2_ref[...] - mu) * inv * g_ref[...] + b_ref[...]


def _k8(h2, sums, sqs, nf2, bn_g, bn_b):
    BN = 1000
    gm = bn_g.reshape(1, G)
    bm = (bn_b - 0.0).reshape(1, G)
    return pl.pallas_call(
        _k8_body,
        grid=(N // BN,),
        in_specs=[
            pl.BlockSpec((BN, G), lambda i: (i, 0)),
            pl.BlockSpec((8, G), lambda i: (0, 0)),
            pl.BlockSpec((8, G), lambda i: (0, 0)),
            pl.BlockSpec((BN, G), lambda i: (i, 0)),
            pl.BlockSpec((1, G), lambda i: (0, 0)),
            pl.BlockSpec((1, G), lambda i: (0, 0)),
        ],
        out_specs=pl.BlockSpec((BN, G), lambda i: (i, 0)),
        out_shape=jax.ShapeDtypeStruct((N, G), jnp.float32),
    )(h2, sums, sqs, nf2, gm, bm)


_USE_SC_GATHER = True
_USE_SC_SCATTER = True


def kernel(node_feats, edge_feats, edge_index, Wpn, bpn, Wpe1, bpe1, Wpe2,
           bpe2, Wet, bet, Wm1, bm1, Wm2, bm2, Wle, ble, Wpn2, bpn2, Wlm1,
           blm1, Wlm2, blm2, bn_g, bn_b):
    src = edge_index[0]
    dst = edge_index[1]
    src2d = src.reshape(NCHUNK, CH)
    dst2d = dst.reshape(NCHUNK, CH)
    z2 = jnp.zeros((N, H), jnp.float32)
    z1 = jnp.zeros((N,), jnp.float32)

    hv_new, P, lv = _k1(node_feats, Wpn, bpn, Wpe1[:DN], Wpe2[:G])
    lv1 = lv[:, 0]

    if _USE_SC_GATHER:
        g1, ld2d, _ = _sc_gather(P, lv1, lv1, src2d, dst2d, dst2d)
        ld = ld2d.reshape(E, 1)
    else:
        g1 = P[src]
        ld = lv1[dst].reshape(E, 1)

    ewa, ewb, ex = _k3(g1, edge_feats, ld, Wpe1[DN:], bpe1, Wpe2[G:], bpe2,
                       Wet, bet)

    if _USE_SC_SCATTER:
        craw_a, craw_b, S = _sc_scatter(ewa, ewb, ex.reshape(NCHUNK, CH),
                                        dst2d, z2, z1)
    else:
        craw_a = jax.ops.segment_sum(ewa, dst, num_segments=N)
        craw_b = jax.ops.segment_sum(ewb, dst, num_segments=N)
        S = jax.ops.segment_sum(ex[:, 0], dst, num_segments=N)

    nf2, u, v, hv2 = _k5(craw_a, craw_b, S.reshape(N, 1), hv_new,
                         Wm1, bm1, Wm2, bm2, Wle, Wpn2, bpn2)

    if _USE_SC_GATHER:
        g2, ud2d, vs2d = _sc_gather(hv2, u[:, 0], v[:, 0], src2d, dst2d,
                                    src2d)
        ud, vs = ud2d.reshape(E, 1), vs2d.reshape(E, 1)
    else:
        g2 = hv2[src]
        ud, vs = u[dst], v[src]

    ga, gb, ex2 = _k6b(g2, ud, vs, ble)

    if _USE_SC_SCATTER:
        traw_a, traw_b, S2 = _sc_scatter(ga, gb, ex2.reshape(NCHUNK, CH),
                                         dst2d, z2, z1)
    else:
        traw_a = jax.ops.segment_sum(ga, dst, num_segments=N)
        traw_b = jax.ops.segment_sum(gb, dst, num_segments=N)
        S2 = jax.ops.segment_sum(ex2[:, 0], dst, num_segments=N)

    h2, sums, sqs = _k7(traw_a, traw_b, S2.reshape(N, 1), nf2,
                        Wlm1, blm1, Wlm2, blm2)
    return _k8(h2, sums, sqs, nf2, bn_g, bn_b)
